# baseline (speedup 1.0000x reference)
import functools

import jax
import jax.numpy as jnp
from jax import lax
from jax.experimental import pallas as pl
from jax.experimental.pallas import tpu as pltpu
from jax.experimental.pallas import tpu_sc as plsc

DIM = 64
_NC = 2
_NS = 16
_NW = _NC * _NS
_NBUF = 8
_RPC = 1


@functools.lru_cache(maxsize=None)
def _make_gather(batch: int, seq: int, V: int):
    rows_per_w = batch // _NW
    n_chunks = rows_per_w // _RPC
    n_outer = n_chunks // _NBUF
    assert batch % _NW == 0 and rows_per_w % (_RPC * _NBUF) == 0

    mesh = plsc.VectorSubcoreMesh(core_axis_name="c", subcore_axis_name="s")

    @functools.partial(
        pl.kernel,
        mesh=mesh,
        out_type=jax.ShapeDtypeStruct((batch, seq, DIM), jnp.float32),
        compiler_params=pltpu.CompilerParams(use_tc_tiling_on_sc=False),
        scratch_types=[
            pltpu.VMEM((_NBUF, seq), jnp.int32),
            pltpu.VMEM((_NBUF, seq, DIM), jnp.float32),
        ]
        + [pltpu.SemaphoreType.DMA] * _NBUF,
    )
    def gather_kernel(x_hbm, table_hbm, out_hbm, idx_v, rows_v, *sems):
        wid = lax.axis_index("s") * _NC + lax.axis_index("c")
        row0 = wid * rows_per_w

        def fire(g, s):
            b = row0 + g * _RPC
            pltpu.sync_copy(x_hbm.at[b], idx_v.at[s])
            pltpu.async_copy(table_hbm.at[idx_v.at[s]], rows_v.at[s], sems[s])

        for s in range(_NBUF):
            fire(s, s)

        def outer(i, carry):
            g0 = i * _NBUF
            for s in range(_NBUF):
                g = g0 + s
                pltpu.make_async_copy(
                    table_hbm.at[idx_v.at[s]], rows_v.at[s], sems[s]
                ).wait()
                b = row0 + g * _RPC
                pltpu.sync_copy(rows_v.at[s], out_hbm.at[b])

                @pl.when(g + _NBUF < n_chunks)
                def _():
                    fire(g + _NBUF, s)

            return carry

        lax.fori_loop(0, n_outer, outer, 0)

    return gather_kernel


def kernel(x, table):
    batch, seq = x.shape
    return _make_gather(batch, seq, table.shape[0])(x, table)


# speedup vs baseline: 1.0205x; 1.0205x over previous
import functools

import jax
import jax.numpy as jnp
from jax import lax
from jax.experimental import pallas as pl
from jax.experimental.pallas import tpu as pltpu
from jax.experimental.pallas import tpu_sc as plsc

DIM = 64
_NC = 2
_NS = 16
_NW = _NC * _NS
_NBUF = 4
_CHUNK = 400


@functools.lru_cache(maxsize=None)
def _make_gather(B: int, V: int):
    assert B % (_NW * _NBUF * _CHUNK) == 0
    b_per_w = B // _NW
    n_chunks = b_per_w // _CHUNK
    n_outer = n_chunks // _NBUF

    mesh = plsc.VectorSubcoreMesh(core_axis_name="c", subcore_axis_name="s")

    @functools.partial(
        pl.kernel,
        mesh=mesh,
        out_type=jax.ShapeDtypeStruct((B, DIM), jnp.float32),
        compiler_params=pltpu.CompilerParams(use_tc_tiling_on_sc=False),
        scratch_types=[
            pltpu.VMEM((_NBUF, _CHUNK), jnp.int32),
            pltpu.VMEM((_NBUF, _CHUNK, DIM), jnp.float32),
        ]
        + [pltpu.SemaphoreType.DMA] * _NBUF,
    )
    def gather_kernel(idx_hbm, table_hbm, out_hbm, idx_v, rows_v, *sems):
        wid = lax.axis_index("s") * _NC + lax.axis_index("c")
        base = wid * b_per_w

        def fire(g, s):
            off = pl.multiple_of(base + g * _CHUNK, 8)
            pltpu.sync_copy(idx_hbm.at[pl.ds(off, _CHUNK)], idx_v.at[s])
            pltpu.async_copy(table_hbm.at[idx_v.at[s]], rows_v.at[s], sems[s])

        for s in range(_NBUF):
            fire(s, s)

        def outer(i, carry):
            g0 = i * _NBUF
            for s in range(_NBUF):
                g = g0 + s
                pltpu.make_async_copy(
                    table_hbm.at[idx_v.at[s]], rows_v.at[s], sems[s]
                ).wait()
                off = pl.multiple_of(base + g * _CHUNK, 8)
                pltpu.sync_copy(rows_v.at[s], out_hbm.at[pl.ds(off, _CHUNK)])

                @pl.when(g + _NBUF < n_chunks)
                def _():
                    fire(g + _NBUF, s)

            return carry

        lax.fori_loop(0, n_outer, outer, 0)

    return gather_kernel


def kernel(x, table):
    batch, seq = x.shape
    B = batch * seq
    gathered = _make_gather(B, table.shape[0])(x.reshape(B), table)
    return gathered.reshape(batch, seq, DIM)
